# Initial kernel scaffold; baseline (speedup 1.0000x reference)
import jax
import jax.numpy as jnp
from jax.experimental import pallas as pl


def kernel(x, router_w, W1, W2):
    raise NotImplementedError("write your pallas kernel here")


# speedup vs baseline: 3.0284x; 3.0284x over previous
import functools

import jax
import jax.numpy as jnp
from jax.experimental import pallas as pl

E = 8
TOP_K = 2


def _moe_body(x_ref, rw_ref, w1_ref, w2_ref, out_ref):
    e = pl.program_id(0)
    xb = x_ref[...]
    logits = jnp.dot(xb, rw_ref[...],
                     preferred_element_type=jnp.float32)
    scores = jax.nn.softmax(logits, axis=-1)
    lane = jax.lax.broadcasted_iota(jnp.int32, scores.shape, 1)
    m1 = jnp.max(scores, axis=-1, keepdims=True)
    idx1 = jnp.min(jnp.where(scores == m1, lane, E), axis=-1, keepdims=True)
    scores2 = jnp.where(lane == idx1, -jnp.inf, scores)
    m2 = jnp.max(scores2, axis=-1, keepdims=True)
    idx2 = jnp.min(jnp.where(scores2 == m2, lane, E), axis=-1, keepdims=True)
    c = (jnp.where(idx1 == e, m1, 0.0)
         + jnp.where(idx2 == e, m2, 0.0))
    h = jnp.dot(xb, w1_ref[0], preferred_element_type=jnp.float32)
    z = jnp.dot(h, w2_ref[0], preferred_element_type=jnp.float32)
    contrib = c * z

    @pl.when(e == 0)
    def _init():
        out_ref[...] = contrib

    @pl.when(e != 0)
    def _acc():
        out_ref[...] += contrib


def kernel(x, router_w, W1, W2):
    sl, bs, hs = x.shape
    T = sl * bs
    dff = W1.shape[2]
    xf = x.reshape(T, hs)
    out = pl.pallas_call(
        _moe_body,
        grid=(E,),
        in_specs=[
            pl.BlockSpec((T, hs), lambda e: (0, 0)),
            pl.BlockSpec((hs, E), lambda e: (0, 0)),
            pl.BlockSpec((1, hs, dff), lambda e: (e, 0, 0)),
            pl.BlockSpec((1, dff, hs), lambda e: (e, 0, 0)),
        ],
        out_specs=pl.BlockSpec((T, hs), lambda e: (0, 0)),
        out_shape=jax.ShapeDtypeStruct((T, hs), jnp.float32),
    )(xf, router_w, W1, W2)
    return out.reshape(sl, bs, hs)
